# baseline (speedup 1.0000x reference)
import functools

import jax
import jax.numpy as jnp
from jax import lax
from jax.experimental import pallas as pl
from jax.experimental.pallas import tpu as pltpu
from jax.experimental.pallas import tpu_sc as plsc

EMBED = 30
D_PAD = 32
B_EDGES = 800000
NUM_CORES = 2
NUM_SUBCORES = 16
CHUNK = 256
C0_CHUNKS = 150
C1_CHUNKS = 50
E0 = C0_CHUNKS * CHUNK
E1 = C1_CHUNKS * CHUNK
SC0_TOTAL = NUM_SUBCORES * E0
B_PAD = NUM_SUBCORES * (E0 + E1)


def _sc_gather_mul(table, src2d, dst2d):
    mesh = plsc.VectorSubcoreMesh(core_axis_name="c", subcore_axis_name="s")

    @functools.partial(
        pl.kernel,
        mesh=mesh,
        out_type=jax.ShapeDtypeStruct((B_PAD, D_PAD), jnp.float32),
        scratch_types=[
            pltpu.VMEM((E0,), jnp.int32),
            pltpu.VMEM((E0,), jnp.int32),
            pltpu.VMEM((CHUNK, D_PAD), jnp.float32),
            pltpu.VMEM((CHUNK, D_PAD), jnp.float32),
            pltpu.VMEM((CHUNK, D_PAD), jnp.float32),
            pltpu.VMEM((CHUNK, D_PAD), jnp.float32),
            pltpu.SemaphoreType.DMA,
            pltpu.SemaphoreType.DMA,
        ],
        compiler_params=pltpu.CompilerParams(use_tc_tiling_on_sc=False),
    )
    def k(table_hbm, src_hbm, dst_hbm, out_hbm, sidx, didx,
          srows0, drows0, srows1, drows1, sem0, sem1):
        cid = lax.axis_index("c")
        sid = lax.axis_index("s")
        base = jnp.where(cid == 0, sid * E0, SC0_TOTAL + sid * E1)
        n_chunks = jnp.where(cid == 0, C0_CHUNKS, C1_CHUNKS)

        @pl.when(cid == 0)
        def _():
            pltpu.sync_copy(src_hbm.at[pl.ds(base, E0)], sidx)
            pltpu.sync_copy(dst_hbm.at[pl.ds(base, E0)], didx)

        @pl.when(cid == 1)
        def _():
            pltpu.sync_copy(src_hbm.at[pl.ds(base, E1)],
                            sidx.at[pl.ds(0, E1)])
            pltpu.sync_copy(dst_hbm.at[pl.ds(base, E1)],
                            didx.at[pl.ds(0, E1)])

        bufs = ((srows0, drows0, sem0), (srows1, drows1, sem1))

        def fire(c, b):
            srows, drows, sem = bufs[b]
            pltpu.async_copy(
                table_hbm.at[sidx.at[pl.ds(c * CHUNK, CHUNK)]], srows, sem)
            pltpu.async_copy(
                table_hbm.at[didx.at[pl.ds(c * CHUNK, CHUNK)]], drows, sem)

        def consume(c, b):
            srows, drows, sem = bufs[b]
            pltpu.make_async_copy(
                table_hbm.at[sidx.at[pl.ds(c * CHUNK, CHUNK)]],
                srows, sem).wait()
            pltpu.make_async_copy(
                table_hbm.at[sidx.at[pl.ds(c * CHUNK, CHUNK)]],
                drows, sem).wait()

            @plsc.parallel_loop(0, CHUNK, 1, unroll=4)
            def _(i):
                a0 = srows[i, pl.ds(0, 16)]
                b0 = drows[i, pl.ds(0, 16)]
                srows[i, pl.ds(0, 16)] = a0 * b0
                a1 = srows[i, pl.ds(16, 16)]
                b1 = drows[i, pl.ds(16, 16)]
                srows[i, pl.ds(16, 16)] = a1 * b1

            pltpu.sync_copy(srows, out_hbm.at[pl.ds(base + c * CHUNK, CHUNK)])

        fire(0, 0)
        fire(1, 1)

        def pair_body(k_it, carry):
            c0 = k_it * 2
            consume(c0, 0)

            @pl.when(c0 + 2 < n_chunks)
            def _():
                fire(c0 + 2, 0)

            consume(c0 + 1, 1)

            @pl.when(c0 + 3 < n_chunks)
            def _():
                fire(c0 + 3, 1)

            return carry

        lax.fori_loop(0, n_chunks // 2, pair_body, 0)

    return k(table, src2d, dst2d)


PACK = 4


def _tc_mlp(e4, w1blk, b1t, wdsel):
    grid = 32
    bt = B_PAD // PACK // grid

    def body(e_ref, w1_ref, b1_ref, wd_ref, out_ref):
        e = e_ref[...]
        h = jnp.dot(e, w1_ref[...], preferred_element_type=jnp.float32)
        h = jnp.maximum(h + b1_ref[...], 0.0)
        out_ref[...] = jnp.dot(h, wd_ref[...],
                               preferred_element_type=jnp.float32)

    return pl.pallas_call(
        body,
        grid=(grid,),
        in_specs=[
            pl.BlockSpec((bt, 128), lambda i: (i, 0)),
            pl.BlockSpec((128, 128), lambda i: (0, 0)),
            pl.BlockSpec((1, 128), lambda i: (0, 0)),
            pl.BlockSpec((128, PACK), lambda i: (0, 0)),
        ],
        out_specs=pl.BlockSpec((bt, PACK), lambda i: (i, 0)),
        out_shape=jax.ShapeDtypeStruct((B_PAD // PACK, PACK), jnp.float32),
    )(e4, w1blk, b1t, wdsel)


def _tc_loss(dmat, labf, bdiff):
    grid = 8
    rows = B_PAD // 128
    br = rows // grid

    def body(d_ref, l_ref, bd_ref, out_ref):
        d = d_ref[...] + bd_ref[0, 0]
        p0 = jnp.where(d >= 0.0,
                       1.0 / (1.0 + jnp.exp(-d)),
                       jnp.exp(d) / (1.0 + jnp.exp(d)))
        p1 = 1.0 - p0
        lse = jnp.log(jnp.exp(p0) + jnp.exp(p1))
        lab = l_ref[...]
        psel = p0 + lab * (1.0 - 2.0 * p0)
        loss_vec = lse - psel
        step = pl.program_id(0)
        row = (lax.broadcasted_iota(jnp.int32, (br, 128), 0) * 128
               + lax.broadcasted_iota(jnp.int32, (br, 128), 1)
               + step * br * 128)
        loss_vec = jnp.where(row < B_EDGES, loss_vec, 0.0)
        s = jnp.sum(loss_vec)

        @pl.when(step == 0)
        def _():
            out_ref[0, 0] = 0.0

        out_ref[0, 0] += s

    return pl.pallas_call(
        body,
        grid=(grid,),
        in_specs=[
            pl.BlockSpec((br, 128), lambda i: (i, 0)),
            pl.BlockSpec((br, 128), lambda i: (i, 0)),
            pl.BlockSpec(memory_space=pltpu.SMEM),
        ],
        out_specs=pl.BlockSpec(memory_space=pltpu.SMEM),
        out_shape=jax.ShapeDtypeStruct((1, 1), jnp.float32),
    )(dmat, labf, bdiff)


def kernel(edges, labels, word_embeddings, W1, b1, W2, b2):
    edges = edges.astype(jnp.int32)
    src = jnp.pad(edges[:, 0], (0, B_PAD - B_EDGES))
    dst = jnp.pad(edges[:, 1], (0, B_PAD - B_EDGES))
    table = jnp.pad(word_embeddings.astype(jnp.float32),
                    ((0, 0), (0, D_PAD - EMBED)))
    eemb = _sc_gather_mul(table, src, dst)

    w1p = jnp.pad(W1, ((0, D_PAD - EMBED), (0, D_PAD - EMBED)))
    b1p = jnp.pad(b1, (0, D_PAD - EMBED))
    wdp = jnp.pad(W2[:, 0] - W2[:, 1], (0, D_PAD - EMBED))
    eye4 = jnp.eye(PACK, dtype=jnp.float32)
    w1blk = jnp.kron(eye4, w1p)
    b1t = jnp.tile(b1p, PACK)[None, :]
    wdsel = jnp.kron(eye4, wdp[:, None])
    e4 = eemb.reshape(B_PAD // PACK, PACK * D_PAD)
    dcol = _tc_mlp(e4, w1blk, b1t, wdsel)
    dmat = dcol.reshape(B_PAD // 128, 128)

    labf = jnp.pad(labels.astype(jnp.float32), (0, B_PAD - B_EDGES)).reshape(
        B_PAD // 128, 128)
    bdiff = (b2[0] - b2[1]).reshape(1, 1)
    out = _tc_loss(dmat, labf, bdiff)
    return out[0, 0] / jnp.float32(B_EDGES)


# speedup vs baseline: 1.1274x; 1.1274x over previous
import functools

import jax
import jax.numpy as jnp
from jax import lax
from jax.experimental import pallas as pl
from jax.experimental.pallas import tpu as pltpu
from jax.experimental.pallas import tpu_sc as plsc

EMBED = 30
D_PAD = 32
B_EDGES = 800000
NUM_CORES = 2
NUM_SUBCORES = 16
NW = NUM_CORES * NUM_SUBCORES
CHUNK = 256
DEPTH = 5
N_CHUNKS = 100
B_PER_W = CHUNK * N_CHUNKS
B_PAD = B_PER_W * NW


def _sc_gather_mul(table, src2d, dst2d):
    mesh = plsc.VectorSubcoreMesh(core_axis_name="c", subcore_axis_name="s")

    scratch = [
        pltpu.VMEM((B_PER_W,), jnp.int32),
        pltpu.VMEM((B_PER_W,), jnp.int32),
    ]
    for _ in range(DEPTH):
        scratch += [
            pltpu.VMEM((CHUNK, D_PAD), jnp.bfloat16),
            pltpu.VMEM((CHUNK, D_PAD), jnp.bfloat16),
            pltpu.VMEM((CHUNK, D_PAD), jnp.bfloat16),
        ]
    scratch += [pltpu.SemaphoreType.DMA] * (2 * DEPTH)

    @functools.partial(
        pl.kernel,
        mesh=mesh,
        out_type=jax.ShapeDtypeStruct((B_PAD, D_PAD), jnp.bfloat16),
        scratch_types=scratch,
        compiler_params=pltpu.CompilerParams(use_tc_tiling_on_sc=False),
    )
    def k(table_hbm, src_hbm, dst_hbm, out_hbm, sidx, didx, *rest):
        rows = rest[:3 * DEPTH]
        sem_g = rest[3 * DEPTH:3 * DEPTH + DEPTH]
        sem_w = rest[3 * DEPTH + DEPTH:]
        bufs = [(rows[3 * b], rows[3 * b + 1], rows[3 * b + 2])
                for b in range(DEPTH)]

        cid = lax.axis_index("c")
        sid = lax.axis_index("s")
        wid = sid * NUM_CORES + cid
        base = wid * B_PER_W

        pltpu.sync_copy(src_hbm.at[pl.ds(base, B_PER_W)], sidx)
        pltpu.sync_copy(dst_hbm.at[pl.ds(base, B_PER_W)], didx)

        def fire(c, b):
            srows, drows, _ = bufs[b]
            pltpu.async_copy(
                table_hbm.at[sidx.at[pl.ds(c * CHUNK, CHUNK)]],
                srows, sem_g[b])
            pltpu.async_copy(
                table_hbm.at[didx.at[pl.ds(c * CHUNK, CHUNK)]],
                drows, sem_g[b])

        def consume(c, b, first, last):
            srows, drows, prod = bufs[b]
            if not first:
                pltpu.make_async_copy(
                    prod, out_hbm.at[pl.ds(base + c * CHUNK, CHUNK)],
                    sem_w[b]).wait()
            pltpu.make_async_copy(
                table_hbm.at[sidx.at[pl.ds(c * CHUNK, CHUNK)]],
                srows, sem_g[b]).wait()
            pltpu.make_async_copy(
                table_hbm.at[sidx.at[pl.ds(c * CHUNK, CHUNK)]],
                drows, sem_g[b]).wait()

            @plsc.parallel_loop(0, CHUNK, 1, unroll=8)
            def _(i):
                prod[i, :] = srows[i, :] * drows[i, :]

            pltpu.async_copy(
                prod, out_hbm.at[pl.ds(base + c * CHUNK, CHUNK)], sem_w[b])
            if not last:
                @pl.when(c + DEPTH < N_CHUNKS)
                def _():
                    fire(c + DEPTH, b)

        for b in range(DEPTH):
            fire(b, b)

        def group_body(g, carry):
            for b in range(DEPTH):
                c = g * DEPTH + b
                consume(c, b, first=False, last=False)
            return carry

        for b in range(DEPTH):
            consume(b, b, first=True, last=False)
        lax.fori_loop(1, N_CHUNKS // DEPTH - 1, group_body, 0)
        for b in range(DEPTH):
            c = N_CHUNKS - DEPTH + b
            consume(c, b, first=False, last=True)
        for b in range(DEPTH):
            c = N_CHUNKS - DEPTH + b
            pltpu.make_async_copy(
                bufs[b][2], out_hbm.at[pl.ds(base + c * CHUNK, CHUNK)],
                sem_w[b]).wait()

    return k(table, src2d, dst2d)


PACK = 4


def _tc_mlp(e4, w1blk, b1t, wdsel):
    grid = 32
    bt = B_PAD // PACK // grid

    def body(e_ref, w1_ref, b1_ref, wd_ref, out_ref):
        e = e_ref[...].astype(jnp.float32)
        h = jnp.dot(e, w1_ref[...], preferred_element_type=jnp.float32)
        h = jnp.maximum(h + b1_ref[...], 0.0)
        out_ref[...] = jnp.dot(h, wd_ref[...],
                               preferred_element_type=jnp.float32)

    return pl.pallas_call(
        body,
        grid=(grid,),
        in_specs=[
            pl.BlockSpec((bt, 128), lambda i: (i, 0)),
            pl.BlockSpec((128, 128), lambda i: (0, 0)),
            pl.BlockSpec((1, 128), lambda i: (0, 0)),
            pl.BlockSpec((128, PACK), lambda i: (0, 0)),
        ],
        out_specs=pl.BlockSpec((bt, PACK), lambda i: (i, 0)),
        out_shape=jax.ShapeDtypeStruct((B_PAD // PACK, PACK), jnp.float32),
    )(e4, w1blk, b1t, wdsel)


def _tc_loss(dmat, labf, bdiff):
    grid = 8
    rows = B_PAD // 128
    br = rows // grid

    def body(d_ref, l_ref, bd_ref, out_ref):
        d = d_ref[...] + bd_ref[0, 0]
        p0 = jnp.where(d >= 0.0,
                       1.0 / (1.0 + jnp.exp(-d)),
                       jnp.exp(d) / (1.0 + jnp.exp(d)))
        p1 = 1.0 - p0
        lse = jnp.log(jnp.exp(p0) + jnp.exp(p1))
        lab = l_ref[...]
        psel = p0 + lab * (1.0 - 2.0 * p0)
        loss_vec = lse - psel
        step = pl.program_id(0)
        row = (lax.broadcasted_iota(jnp.int32, (br, 128), 0) * 128
               + lax.broadcasted_iota(jnp.int32, (br, 128), 1)
               + step * br * 128)
        loss_vec = jnp.where(row < B_EDGES, loss_vec, 0.0)
        s = jnp.sum(loss_vec)

        @pl.when(step == 0)
        def _():
            out_ref[0, 0] = 0.0

        out_ref[0, 0] += s

    return pl.pallas_call(
        body,
        grid=(grid,),
        in_specs=[
            pl.BlockSpec((br, 128), lambda i: (i, 0)),
            pl.BlockSpec((br, 128), lambda i: (i, 0)),
            pl.BlockSpec(memory_space=pltpu.SMEM),
        ],
        out_specs=pl.BlockSpec(memory_space=pltpu.SMEM),
        out_shape=jax.ShapeDtypeStruct((1, 1), jnp.float32),
    )(dmat, labf, bdiff)


def kernel(edges, labels, word_embeddings, W1, b1, W2, b2):
    edges = edges.astype(jnp.int32)
    src = jnp.pad(edges[:, 0], (0, B_PAD - B_EDGES))
    dst = jnp.pad(edges[:, 1], (0, B_PAD - B_EDGES))
    table = jnp.pad(word_embeddings.astype(jnp.float32),
                    ((0, 0), (0, D_PAD - EMBED))).astype(jnp.bfloat16)
    eemb = _sc_gather_mul(table, src, dst)

    w1p = jnp.pad(W1, ((0, D_PAD - EMBED), (0, D_PAD - EMBED)))
    b1p = jnp.pad(b1, (0, D_PAD - EMBED))
    wdp = jnp.pad(W2[:, 0] - W2[:, 1], (0, D_PAD - EMBED))
    eye4 = jnp.eye(PACK, dtype=jnp.float32)
    w1blk = jnp.kron(eye4, w1p)
    b1t = jnp.tile(b1p, PACK)[None, :]
    wdsel = jnp.kron(eye4, wdp[:, None])
    e4 = eemb.reshape(B_PAD // PACK, PACK * D_PAD)
    dcol = _tc_mlp(e4, w1blk, b1t, wdsel)
    dmat = dcol.reshape(B_PAD // 128, 128)

    labf = jnp.pad(labels.astype(jnp.float32), (0, B_PAD - B_EDGES)).reshape(
        B_PAD // 128, 128)
    bdiff = (b2[0] - b2[1]).reshape(1, 1)
    out = _tc_loss(dmat, labf, bdiff)
    return out[0, 0] / jnp.float32(B_EDGES)


# speedup vs baseline: 1.1701x; 1.0378x over previous
import functools

import jax
import jax.numpy as jnp
from jax import lax
from jax.experimental import pallas as pl
from jax.experimental.pallas import tpu as pltpu
from jax.experimental.pallas import tpu_sc as plsc

EMBED = 30
D_PAD = 32
B_EDGES = 800000
NUM_CORES = 2
NUM_SUBCORES = 16
NW = NUM_CORES * NUM_SUBCORES
CHUNK = 256
DEPTH = 5
N_CHUNKS = 100
B_PER_W = CHUNK * N_CHUNKS
B_PAD = B_PER_W * NW
PACK = 4


def _sc_gather_mul(table, src2d, dst2d):
    mesh = plsc.VectorSubcoreMesh(core_axis_name="c", subcore_axis_name="s")

    scratch = [
        pltpu.VMEM((B_PER_W,), jnp.int32),
        pltpu.VMEM((B_PER_W,), jnp.int32),
    ]
    for _ in range(DEPTH):
        scratch += [
            pltpu.VMEM((CHUNK, D_PAD), jnp.bfloat16),
            pltpu.VMEM((CHUNK, D_PAD), jnp.bfloat16),
            pltpu.VMEM((CHUNK // PACK, 128), jnp.bfloat16),
        ]
    scratch += [pltpu.SemaphoreType.DMA] * (2 * DEPTH)

    @functools.partial(
        pl.kernel,
        mesh=mesh,
        out_type=jax.ShapeDtypeStruct((B_PAD // PACK, PACK * D_PAD),
                                      jnp.bfloat16),
        scratch_types=scratch,
        compiler_params=pltpu.CompilerParams(use_tc_tiling_on_sc=False),
    )
    def k(table_hbm, src_hbm, dst_hbm, out_hbm, sidx, didx, *rest):
        rows = rest[:3 * DEPTH]
        sem_g = rest[3 * DEPTH:3 * DEPTH + DEPTH]
        sem_w = rest[3 * DEPTH + DEPTH:]
        bufs = [(rows[3 * b], rows[3 * b + 1], rows[3 * b + 2])
                for b in range(DEPTH)]

        cid = lax.axis_index("c")
        sid = lax.axis_index("s")
        wid = sid * NUM_CORES + cid
        base = wid * B_PER_W

        pltpu.sync_copy(src_hbm.at[pl.ds(base, B_PER_W)], sidx)
        pltpu.sync_copy(dst_hbm.at[pl.ds(base, B_PER_W)], didx)

        def fire(c, b):
            srows, drows, _ = bufs[b]
            pltpu.async_copy(
                table_hbm.at[sidx.at[pl.ds(c * CHUNK, CHUNK)]],
                srows, sem_g[b])
            pltpu.async_copy(
                table_hbm.at[didx.at[pl.ds(c * CHUNK, CHUNK)]],
                drows, sem_g[b])

        def consume(c, b, first, last):
            srows, drows, prod = bufs[b]
            obase = (base // PACK) + c * (CHUNK // PACK)
            if not first:
                pltpu.make_async_copy(
                    prod, out_hbm.at[pl.ds(obase, CHUNK // PACK)],
                    sem_w[b]).wait()
            pltpu.make_async_copy(
                table_hbm.at[sidx.at[pl.ds(c * CHUNK, CHUNK)]],
                srows, sem_g[b]).wait()
            pltpu.make_async_copy(
                table_hbm.at[sidx.at[pl.ds(c * CHUNK, CHUNK)]],
                drows, sem_g[b]).wait()

            @plsc.parallel_loop(0, CHUNK // PACK, 1, unroll=4)
            def _(j):
                for g in range(PACK):
                    prod[j, pl.ds(g * D_PAD, D_PAD)] = (
                        srows[PACK * j + g, :] * drows[PACK * j + g, :])

            pltpu.async_copy(
                prod, out_hbm.at[pl.ds(obase, CHUNK // PACK)], sem_w[b])
            if not last:
                @pl.when(c + DEPTH < N_CHUNKS)
                def _():
                    fire(c + DEPTH, b)

        for b in range(DEPTH):
            fire(b, b)

        def group_body(g, carry):
            for b in range(DEPTH):
                c = g * DEPTH + b
                consume(c, b, first=False, last=False)
            return carry

        for b in range(DEPTH):
            consume(b, b, first=True, last=False)
        lax.fori_loop(1, N_CHUNKS // DEPTH - 1, group_body, 0)
        for b in range(DEPTH):
            c = N_CHUNKS - DEPTH + b
            consume(c, b, first=False, last=True)
        for b in range(DEPTH):
            c = N_CHUNKS - DEPTH + b
            pltpu.make_async_copy(
                bufs[b][2],
                out_hbm.at[pl.ds((base // PACK) + c * (CHUNK // PACK),
                                 CHUNK // PACK)],
                sem_w[b]).wait()

    return k(table, src2d, dst2d)


def _tc_mlp_loss(e4, labt, w1blk, b1t, wdsel, bdiff):
    grid = 32
    bt = B_PAD // PACK // grid

    def body(e_ref, l_ref, w1_ref, b1_ref, wd_ref, bd_ref, out_ref):
        e = e_ref[...].astype(jnp.float32)
        h = jnp.dot(e, w1_ref[...], preferred_element_type=jnp.float32)
        h = jnp.maximum(h + b1_ref[...], 0.0)
        d4 = jnp.dot(h, wd_ref[...], preferred_element_type=jnp.float32)
        d = d4.T + bd_ref[0, 0]
        p0 = jnp.where(d >= 0.0,
                       1.0 / (1.0 + jnp.exp(-d)),
                       jnp.exp(d) / (1.0 + jnp.exp(d)))
        p1 = 1.0 - p0
        lse = jnp.log(jnp.exp(p0) + jnp.exp(p1))
        lab = l_ref[...]
        psel = p0 + lab * (1.0 - 2.0 * p0)
        loss_vec = lse - psel
        step = pl.program_id(0)
        edge = (lax.broadcasted_iota(jnp.int32, (PACK, bt), 0)
                + lax.broadcasted_iota(jnp.int32, (PACK, bt), 1) * PACK
                + step * bt * PACK)
        loss_vec = jnp.where(edge < B_EDGES, loss_vec, 0.0)
        s = jnp.sum(loss_vec)

        @pl.when(step == 0)
        def _():
            out_ref[0, 0] = 0.0

        out_ref[0, 0] += s

    return pl.pallas_call(
        body,
        grid=(grid,),
        in_specs=[
            pl.BlockSpec((bt, 128), lambda i: (i, 0)),
            pl.BlockSpec((PACK, bt), lambda i: (0, i)),
            pl.BlockSpec((128, 128), lambda i: (0, 0)),
            pl.BlockSpec((1, 128), lambda i: (0, 0)),
            pl.BlockSpec((128, PACK), lambda i: (0, 0)),
            pl.BlockSpec(memory_space=pltpu.SMEM),
        ],
        out_specs=pl.BlockSpec(memory_space=pltpu.SMEM),
        out_shape=jax.ShapeDtypeStruct((1, 1), jnp.float32),
    )(e4, labt, w1blk, b1t, wdsel, bdiff)


def kernel(edges, labels, word_embeddings, W1, b1, W2, b2):
    edges = edges.astype(jnp.int32)
    src = jnp.pad(edges[:, 0], (0, B_PAD - B_EDGES))
    dst = jnp.pad(edges[:, 1], (0, B_PAD - B_EDGES))
    table = jnp.pad(word_embeddings.astype(jnp.bfloat16),
                    ((0, 0), (0, D_PAD - EMBED)))
    e4 = _sc_gather_mul(table, src, dst)

    w1p = jnp.pad(W1, ((0, D_PAD - EMBED), (0, D_PAD - EMBED)))
    b1p = jnp.pad(b1, (0, D_PAD - EMBED))
    wdp = jnp.pad(W2[:, 0] - W2[:, 1], (0, D_PAD - EMBED))
    eye4 = jnp.eye(PACK, dtype=jnp.float32)
    w1blk = jnp.kron(eye4, w1p)
    b1t = jnp.tile(b1p, PACK)[None, :]
    wdsel = jnp.kron(eye4, wdp[:, None])

    labt = jnp.pad(labels.astype(jnp.float32), (0, B_PAD - B_EDGES)).reshape(
        B_PAD // PACK, PACK).T
    bdiff = (b2[0] - b2[1]).reshape(1, 1)
    out = _tc_mlp_loss(e4, labt, w1blk, b1t, wdsel, bdiff)
    return out[0, 0] / jnp.float32(B_EDGES)


# speedup vs baseline: 2.1068x; 1.8006x over previous
import functools

import jax
import jax.numpy as jnp
from jax import lax
from jax.experimental import pallas as pl
from jax.experimental.pallas import tpu as pltpu
from jax.experimental.pallas import tpu_sc as plsc

EMBED = 30
D_PAD = 32
B_EDGES = 800000
NUM_CORES = 2
NUM_SUBCORES = 16
CHUNK = 256
DEPTH = 4
C0_CHUNKS = 156
C1_CHUNKS = 44
E0 = C0_CHUNKS * CHUNK
E1 = C1_CHUNKS * CHUNK
SC0_TOTAL = NUM_SUBCORES * E0
B_PAD = NUM_SUBCORES * (E0 + E1)
EPR = 8
OUT_ROWS = B_PAD // EPR
CHUNK_ROWS = CHUNK // EPR


def _sc_gather_mul(table, src, dst):
    mesh = plsc.VectorSubcoreMesh(core_axis_name="c", subcore_axis_name="s")

    scratch = [
        pltpu.VMEM((E0,), jnp.int32),
        pltpu.VMEM((E0,), jnp.int32),
    ]
    for _ in range(DEPTH):
        scratch += [
            pltpu.VMEM((CHUNK, D_PAD), jnp.bfloat16),
            pltpu.VMEM((CHUNK, D_PAD), jnp.bfloat16),
            pltpu.VMEM((CHUNK_ROWS, 128), jnp.int32),
        ]
    scratch += [pltpu.SemaphoreType.DMA] * (2 * DEPTH)

    @functools.partial(
        pl.kernel,
        mesh=mesh,
        out_type=jax.ShapeDtypeStruct((OUT_ROWS, 128), jnp.int32),
        scratch_types=scratch,
        compiler_params=pltpu.CompilerParams(use_tc_tiling_on_sc=False,
                                             needs_layout_passes=False),
    )
    def k(table_hbm, src_hbm, dst_hbm, out_hbm, sidx, didx, *rest):
        rows = rest[:3 * DEPTH]
        sem_g = rest[3 * DEPTH:3 * DEPTH + DEPTH]
        sem_w = rest[3 * DEPTH + DEPTH:]
        bufs = [(rows[3 * b], rows[3 * b + 1], rows[3 * b + 2])
                for b in range(DEPTH)]

        cid = lax.axis_index("c")
        sid = lax.axis_index("s")
        base = jnp.where(cid == 0, sid * E0, SC0_TOTAL + sid * E1)
        n_chunks = jnp.where(cid == 0, C0_CHUNKS, C1_CHUNKS)

        @pl.when(cid == 0)
        def _():
            pltpu.sync_copy(src_hbm.at[pl.ds(base, E0)], sidx)
            pltpu.sync_copy(dst_hbm.at[pl.ds(base, E0)], didx)

        @pl.when(cid == 1)
        def _():
            pltpu.sync_copy(src_hbm.at[pl.ds(base, E1)],
                            sidx.at[pl.ds(0, E1)])
            pltpu.sync_copy(dst_hbm.at[pl.ds(base, E1)],
                            didx.at[pl.ds(0, E1)])

        def fire(c, b):
            srows, drows, _ = bufs[b]
            pltpu.async_copy(
                table_hbm.at[sidx.at[pl.ds(c * CHUNK, CHUNK)]],
                srows, sem_g[b])
            pltpu.async_copy(
                table_hbm.at[didx.at[pl.ds(c * CHUNK, CHUNK)]],
                drows, sem_g[b])

        def consume(c, b, g_it):
            srows, drows, prod = bufs[b]
            obase = (base // EPR) + c * CHUNK_ROWS

            @pl.when(g_it > 0)
            def _():
                pltpu.make_async_copy(
                    prod, out_hbm.at[pl.ds(obase, CHUNK_ROWS)],
                    sem_w[b]).wait()

            pltpu.make_async_copy(
                table_hbm.at[sidx.at[pl.ds(c * CHUNK, CHUNK)]],
                srows, sem_g[b]).wait()
            pltpu.make_async_copy(
                table_hbm.at[sidx.at[pl.ds(c * CHUNK, CHUNK)]],
                drows, sem_g[b]).wait()

            @plsc.parallel_loop(0, CHUNK, 1, unroll=8)
            def _(i):
                p = srows[i, :] * drows[i, :]
                pi = plsc.bitcast(p, jnp.int32)
                prod[i // EPR, pl.ds((i % EPR) * 16, 16)] = pi

            pltpu.async_copy(
                prod, out_hbm.at[pl.ds(obase, CHUNK_ROWS)], sem_w[b])

            @pl.when(c + DEPTH < n_chunks)
            def _():
                fire(c + DEPTH, b)

        for b in range(DEPTH):
            fire(b, b)

        def group_body(g_it, carry):
            for b in range(DEPTH):
                consume(g_it * DEPTH + b, b, g_it)
            return carry

        lax.fori_loop(0, n_chunks // DEPTH, group_body, 0)

        for b in range(DEPTH):
            c = n_chunks - DEPTH + b
            pltpu.make_async_copy(
                bufs[b][2],
                out_hbm.at[pl.ds((base // EPR) + c * CHUNK_ROWS,
                                 CHUNK_ROWS)],
                sem_w[b]).wait()

    return k(table, src, dst)


def _tc_mlp_loss(e8, labt, w1e, w1o, b1t, wdsel, bdiff):
    grid = 32
    bt = OUT_ROWS // grid

    def body(e_ref, l_ref, w1e_ref, w1o_ref, b1_ref, wd_ref, bd_ref,
             out_ref):
        ei = e_ref[...]
        evens = lax.bitcast_convert_type(ei << 16, jnp.float32)
        odds = lax.bitcast_convert_type(
            ei & jnp.int32(-65536), jnp.float32)
        h = (jnp.dot(evens, w1e_ref[...],
                     preferred_element_type=jnp.float32)
             + jnp.dot(odds, w1o_ref[...],
                       preferred_element_type=jnp.float32))
        h = jnp.maximum(h + b1_ref[...], 0.0)
        d8 = jnp.dot(h, wd_ref[...], preferred_element_type=jnp.float32)
        d = d8.T + bd_ref[0, 0]
        p0 = jnp.where(d >= 0.0,
                       1.0 / (1.0 + jnp.exp(-d)),
                       jnp.exp(d) / (1.0 + jnp.exp(d)))
        p1 = 1.0 - p0
        lse = jnp.log(jnp.exp(p0) + jnp.exp(p1))
        lab = l_ref[...]
        psel = p0 + lab * (1.0 - 2.0 * p0)
        loss_vec = lse - psel
        step = pl.program_id(0)
        edge = (lax.broadcasted_iota(jnp.int32, (EPR, bt), 0)
                + lax.broadcasted_iota(jnp.int32, (EPR, bt), 1) * EPR
                + step * bt * EPR)
        loss_vec = jnp.where(edge < B_EDGES, loss_vec, 0.0)
        s = jnp.sum(loss_vec)

        @pl.when(step == 0)
        def _():
            out_ref[0, 0] = 0.0

        out_ref[0, 0] += s

    return pl.pallas_call(
        body,
        grid=(grid,),
        in_specs=[
            pl.BlockSpec((bt, 128), lambda i: (i, 0)),
            pl.BlockSpec((EPR, bt), lambda i: (0, i)),
            pl.BlockSpec((128, 256), lambda i: (0, 0)),
            pl.BlockSpec((128, 256), lambda i: (0, 0)),
            pl.BlockSpec((1, 256), lambda i: (0, 0)),
            pl.BlockSpec((256, EPR), lambda i: (0, 0)),
            pl.BlockSpec(memory_space=pltpu.SMEM),
        ],
        out_specs=pl.BlockSpec(memory_space=pltpu.SMEM),
        out_shape=jax.ShapeDtypeStruct((1, 1), jnp.float32),
    )(e8, labt, w1e, w1o, b1t, wdsel, bdiff)


def kernel(edges, labels, word_embeddings, W1, b1, W2, b2):
    edges = edges.astype(jnp.int32)
    src = jnp.pad(edges[:, 0], (0, B_PAD - B_EDGES))
    dst = jnp.pad(edges[:, 1], (0, B_PAD - B_EDGES))
    table = jnp.pad(word_embeddings.astype(jnp.bfloat16),
                    ((0, 0), (0, D_PAD - EMBED)))
    e8 = _sc_gather_mul(table, src, dst)

    w1p = jnp.pad(W1, ((0, D_PAD - EMBED), (0, D_PAD - EMBED)))
    b1p = jnp.pad(b1, (0, D_PAD - EMBED))
    wdp = jnp.pad(W2[:, 0] - W2[:, 1], (0, D_PAD - EMBED))
    eye8 = jnp.eye(EPR, dtype=jnp.float32)
    w1e = jnp.kron(eye8, w1p[0::2, :])
    w1o = jnp.kron(eye8, w1p[1::2, :])
    b1t = jnp.tile(b1p, EPR)[None, :]
    wdsel = jnp.kron(eye8, wdp[:, None])

    labpad = jnp.pad(labels.astype(jnp.float32), (0, B_PAD - B_EDGES))
    labt = jnp.stack([labpad[g::EPR] for g in range(EPR)], axis=0)
    bdiff = (b2[0] - b2[1]).reshape(1, 1)
    out = _tc_mlp_loss(e8, labt, w1e, w1o, b1t, wdsel, bdiff)
    return out[0, 0] / jnp.float32(B_EDGES)


# speedup vs baseline: 2.1869x; 1.0380x over previous
import functools

import jax
import jax.numpy as jnp
from jax import lax
from jax.experimental import pallas as pl
from jax.experimental.pallas import tpu as pltpu
from jax.experimental.pallas import tpu_sc as plsc

EMBED = 30
D_PAD = 32
B_EDGES = 800000
NUM_CORES = 2
NUM_SUBCORES = 16
CHUNK = 256
DEPTH = 4
C0_CHUNKS = 156
C1_CHUNKS = 44
E0 = C0_CHUNKS * CHUNK
E1 = C1_CHUNKS * CHUNK
SC0_TOTAL = NUM_SUBCORES * E0
B_PAD = NUM_SUBCORES * (E0 + E1)
EPR = 8
OUT_ROWS = B_PAD // EPR
CHUNK_ROWS = CHUNK // EPR


def _sc_gather_mul(table, idx2):
    mesh = plsc.VectorSubcoreMesh(core_axis_name="c", subcore_axis_name="s")

    scratch = [
        pltpu.VMEM((2 * E0,), jnp.int32),
    ]
    for _ in range(DEPTH):
        scratch += [
            pltpu.VMEM((2 * CHUNK, D_PAD), jnp.bfloat16),
            pltpu.VMEM((CHUNK_ROWS, 128), jnp.int32),
        ]
    scratch += [pltpu.SemaphoreType.DMA] * (2 * DEPTH)

    @functools.partial(
        pl.kernel,
        mesh=mesh,
        out_type=jax.ShapeDtypeStruct((OUT_ROWS, 128), jnp.int32),
        scratch_types=scratch,
        compiler_params=pltpu.CompilerParams(use_tc_tiling_on_sc=False,
                                             needs_layout_passes=False),
    )
    def k(table_hbm, idx_hbm, out_hbm, cidx, *rest):
        rows = rest[:2 * DEPTH]
        sem_g = rest[2 * DEPTH:2 * DEPTH + DEPTH]
        sem_w = rest[2 * DEPTH + DEPTH:]
        bufs = [(rows[2 * b], rows[2 * b + 1]) for b in range(DEPTH)]

        cid = lax.axis_index("c")
        sid = lax.axis_index("s")
        base = jnp.where(cid == 0, sid * E0, SC0_TOTAL + sid * E1)
        n_chunks = jnp.where(cid == 0, C0_CHUNKS, C1_CHUNKS)

        @pl.when(cid == 0)
        def _():
            pltpu.sync_copy(idx_hbm.at[pl.ds(2 * base, 2 * E0)], cidx)

        @pl.when(cid == 1)
        def _():
            pltpu.sync_copy(idx_hbm.at[pl.ds(2 * base, 2 * E1)],
                            cidx.at[pl.ds(0, 2 * E1)])

        def fire(c, b):
            rbuf, _ = bufs[b]
            pltpu.async_copy(
                table_hbm.at[cidx.at[pl.ds(c * 2 * CHUNK, 2 * CHUNK)]],
                rbuf, sem_g[b])

        def consume(c, b, g_it):
            rbuf, prod = bufs[b]
            obase = (base // EPR) + c * CHUNK_ROWS

            @pl.when(g_it > 0)
            def _():
                pltpu.make_async_copy(
                    prod, out_hbm.at[pl.ds(obase, CHUNK_ROWS)],
                    sem_w[b]).wait()

            pltpu.make_async_copy(
                table_hbm.at[cidx.at[pl.ds(c * 2 * CHUNK, 2 * CHUNK)]],
                rbuf, sem_g[b]).wait()

            @plsc.parallel_loop(0, CHUNK, 1, unroll=8)
            def _(i):
                p = rbuf[i, :] * rbuf[CHUNK + i, :]
                pi = plsc.bitcast(p, jnp.int32)
                prod[i // EPR, pl.ds((i % EPR) * 16, 16)] = pi

            pltpu.async_copy(
                prod, out_hbm.at[pl.ds(obase, CHUNK_ROWS)], sem_w[b])

            @pl.when(c + DEPTH < n_chunks)
            def _():
                fire(c + DEPTH, b)

        for b in range(DEPTH):
            fire(b, b)

        def group_body(g_it, carry):
            for b in range(DEPTH):
                consume(g_it * DEPTH + b, b, g_it)
            return carry

        lax.fori_loop(0, n_chunks // DEPTH, group_body, 0)

        for b in range(DEPTH):
            c = n_chunks - DEPTH + b
            pltpu.make_async_copy(
                bufs[b][1],
                out_hbm.at[pl.ds((base // EPR) + c * CHUNK_ROWS,
                                 CHUNK_ROWS)],
                sem_w[b]).wait()

    return k(table, idx2)


def _tc_mlp_loss(e8, labt, w1e, w1o, b1t, wdsel, bdiff):
    grid = 32
    bt = OUT_ROWS // grid

    def body(e_ref, l_ref, w1e_ref, w1o_ref, b1_ref, wd_ref, bd_ref,
             out_ref):
        ei = e_ref[...]
        evens = lax.bitcast_convert_type(ei << 16, jnp.float32)
        odds = lax.bitcast_convert_type(
            ei & jnp.int32(-65536), jnp.float32)
        h = (jnp.dot(evens, w1e_ref[...],
                     preferred_element_type=jnp.float32)
             + jnp.dot(odds, w1o_ref[...],
                       preferred_element_type=jnp.float32))
        h = jnp.maximum(h + b1_ref[...], 0.0)
        d8 = jnp.dot(h, wd_ref[...], preferred_element_type=jnp.float32)
        d = d8.T + bd_ref[0, 0]
        p0 = jnp.where(d >= 0.0,
                       1.0 / (1.0 + jnp.exp(-d)),
                       jnp.exp(d) / (1.0 + jnp.exp(d)))
        p1 = 1.0 - p0
        lse = jnp.log(jnp.exp(p0) + jnp.exp(p1))
        lab = l_ref[...]
        psel = p0 + lab * (1.0 - 2.0 * p0)
        loss_vec = lse - psel
        step = pl.program_id(0)
        edge = (lax.broadcasted_iota(jnp.int32, (EPR, bt), 0)
                + lax.broadcasted_iota(jnp.int32, (EPR, bt), 1) * EPR
                + step * bt * EPR)
        loss_vec = jnp.where(edge < B_EDGES, loss_vec, 0.0)
        s = jnp.sum(loss_vec)

        @pl.when(step == 0)
        def _():
            out_ref[0, 0] = 0.0

        out_ref[0, 0] += s

    return pl.pallas_call(
        body,
        grid=(grid,),
        in_specs=[
            pl.BlockSpec((bt, 128), lambda i: (i, 0)),
            pl.BlockSpec((EPR, bt), lambda i: (0, i)),
            pl.BlockSpec((128, 256), lambda i: (0, 0)),
            pl.BlockSpec((128, 256), lambda i: (0, 0)),
            pl.BlockSpec((1, 256), lambda i: (0, 0)),
            pl.BlockSpec((256, EPR), lambda i: (0, 0)),
            pl.BlockSpec(memory_space=pltpu.SMEM),
        ],
        out_specs=pl.BlockSpec(memory_space=pltpu.SMEM),
        out_shape=jax.ShapeDtypeStruct((1, 1), jnp.float32),
    )(e8, labt, w1e, w1o, b1t, wdsel, bdiff)


def kernel(edges, labels, word_embeddings, W1, b1, W2, b2):
    edges = edges.astype(jnp.int32)
    ep = jnp.pad(edges, ((0, B_PAD - B_EDGES), (0, 0)))
    idx2 = jnp.transpose(ep.reshape(B_PAD // CHUNK, CHUNK, 2),
                         (0, 2, 1)).reshape(2 * B_PAD)
    table = jnp.pad(word_embeddings.astype(jnp.bfloat16),
                    ((0, 0), (0, D_PAD - EMBED)))
    e8 = _sc_gather_mul(table, idx2)

    w1p = jnp.pad(W1, ((0, D_PAD - EMBED), (0, D_PAD - EMBED)))
    b1p = jnp.pad(b1, (0, D_PAD - EMBED))
    wdp = jnp.pad(W2[:, 0] - W2[:, 1], (0, D_PAD - EMBED))
    eye8 = jnp.eye(EPR, dtype=jnp.float32)
    w1e = jnp.kron(eye8, w1p[0::2, :])
    w1o = jnp.kron(eye8, w1p[1::2, :])
    b1t = jnp.tile(b1p, EPR)[None, :]
    wdsel = jnp.kron(eye8, wdp[:, None])

    labpad = jnp.pad(labels.astype(jnp.float32), (0, B_PAD - B_EDGES))
    labt = jnp.stack([labpad[g::EPR] for g in range(EPR)], axis=0)
    bdiff = (b2[0] - b2[1]).reshape(1, 1)
    out = _tc_mlp_loss(e8, labt, w1e, w1o, b1t, wdsel, bdiff)
    return out[0, 0] / jnp.float32(B_EDGES)
